# baseline (speedup 1.0000x reference)
import functools

import jax
import jax.numpy as jnp
from jax.experimental import pallas as pl
from jax.experimental.pallas import tpu as pltpu

B, S, D, E = 2, 2048, 768, 8
T = B * S
EPAD = 128


def _linear_body(x_ref, w_ref, b_ref, o_ref):
    o_ref[...] = (
        jnp.dot(x_ref[...], w_ref[...], preferred_element_type=jnp.float32)
        + b_ref[...]
    )


def _linear(x, w, b, tile=256):
    return pl.pallas_call(
        _linear_body,
        grid=(T // tile,),
        in_specs=[
            pl.BlockSpec((tile, D), lambda t: (t, 0)),
            pl.BlockSpec((D, D), lambda t: (0, 0)),
            pl.BlockSpec((1, D), lambda t: (0, 0)),
        ],
        out_specs=pl.BlockSpec((tile, D), lambda t: (t, 0)),
        out_shape=jax.ShapeDtypeStruct((T, D), jnp.float32),
    )(x, w, b.reshape(1, D))


def _moe_body(x_ref, wg_ref, We_ref, be_ref, o_ref, gate_scr, idx_scr):
    e = pl.program_id(1)

    @pl.when(e == 0)
    def _():
        logits = jnp.dot(x_ref[...], wg_ref[...], preferred_element_type=jnp.float32)
        col = jax.lax.broadcasted_iota(jnp.int32, logits.shape, 1)
        valid = col < E
        logits = jnp.where(valid, logits, -jnp.inf)
        m = jnp.max(logits, axis=1, keepdims=True)
        p = jnp.exp(logits - m)
        s = jnp.sum(p, axis=1, keepdims=True)
        gate_scr[...] = jnp.max(p, axis=1, keepdims=True) / s
        is_max = (logits == m) & valid
        idx_scr[...] = jnp.min(jnp.where(is_max, col, EPAD), axis=1, keepdims=True)
        o_ref[...] = jnp.zeros_like(o_ref)

    w = jnp.where(idx_scr[...] == e, gate_scr[...], 0.0)
    y = (
        jnp.dot(x_ref[...], We_ref[0], preferred_element_type=jnp.float32)
        + be_ref[0]
    )
    o_ref[...] += w * y


def _moe(x, wg, We, be, tile=256):
    wgp = jnp.pad(wg, ((0, 0), (0, EPAD - E)))
    return pl.pallas_call(
        _moe_body,
        grid=(T // tile, E),
        in_specs=[
            pl.BlockSpec((tile, D), lambda t, e: (t, 0)),
            pl.BlockSpec((D, EPAD), lambda t, e: (0, 0)),
            pl.BlockSpec((1, D, D), lambda t, e: (e, 0, 0)),
            pl.BlockSpec((1, 1, D), lambda t, e: (e, 0, 0)),
        ],
        out_specs=pl.BlockSpec((tile, D), lambda t, e: (t, 0)),
        out_shape=jax.ShapeDtypeStruct((T, D), jnp.float32),
        scratch_shapes=[
            pltpu.VMEM((tile, 1), jnp.float32),
            pltpu.VMEM((tile, 1), jnp.int32),
        ],
    )(x, wgp, We, be.reshape(E, 1, D))


def _loss_body(h_ref, o_ref, y_ref, loss_ref):
    acc = h_ref[...] + o_ref[...]
    s0 = jnp.sum(acc[:S], axis=0, keepdims=True)
    s1 = jnp.sum(acc[S:], axis=0, keepdims=True)
    emb = jnp.concatenate([s0, s1], axis=0) / S
    m = jnp.max(emb, axis=1, keepdims=True)
    lse = m + jnp.log(jnp.sum(jnp.exp(emb - m), axis=1, keepdims=True))
    logp = emb - lse
    col = jax.lax.broadcasted_iota(jnp.int32, logp.shape, 1)
    v0 = jnp.sum(jnp.where(col[0:1] == y_ref[0], logp[0:1], 0.0))
    v1 = jnp.sum(jnp.where(col[1:2] == y_ref[1], logp[1:2], 0.0))
    loss_ref[0, 0] = -(v0 + v1) / B


def _loss(hidden, out2, y):
    out = pl.pallas_call(
        _loss_body,
        in_specs=[
            pl.BlockSpec((T, D), lambda: (0, 0)),
            pl.BlockSpec((T, D), lambda: (0, 0)),
            pl.BlockSpec(memory_space=pltpu.SMEM),
        ],
        out_specs=pl.BlockSpec(memory_space=pltpu.SMEM),
        out_shape=jax.ShapeDtypeStruct((1, 1), jnp.float32),
    )(hidden, out2, y)
    return out.reshape(())


def kernel(x, y, W1, b1, wg2, We2, be2, wg3, We3, be3):
    xt = x.reshape(T, D)
    hidden = _linear(xt, W1.T, b1)
    out = _moe(hidden, wg2, We2, be2)
    out = _moe(out, wg3, We3, be3)
    return _loss(hidden, out, y.astype(jnp.int32))


# speedup vs baseline: 1.3597x; 1.3597x over previous
import functools

import jax
import jax.numpy as jnp
from jax import lax
from jax.experimental import pallas as pl
from jax.experimental.pallas import tpu as pltpu
from jax.experimental.pallas import tpu_sc as plsc

B, S, D, E = 2, 2048, 768, 8
T = B * S
EPAD = 128
TILE = 128
TILE_SHIFT = 7
TP = T + E * TILE
XT = TP // TILE
NW = 16
CH = T // NW
HALF = CH // 2


def _router(logits):
    col = jax.lax.broadcasted_iota(jnp.int32, logits.shape, 1)
    valid = col < E
    logits = jnp.where(valid, logits, -jnp.inf)
    m = jnp.max(logits, axis=1, keepdims=True)
    p = jnp.exp(logits - m)
    s = jnp.sum(p, axis=1, keepdims=True)
    gate = jnp.max(p, axis=1, keepdims=True) / s
    is_max = (logits == m) & valid
    idx = jnp.min(jnp.where(is_max, col, EPAD), axis=1, keepdims=True)
    return gate, idx


def _dense_router_body(x_ref, w_ref, b_ref, wg_ref, h_ref, g_ref, i_ref):
    h = (
        jnp.dot(x_ref[...], w_ref[...], preferred_element_type=jnp.float32)
        + b_ref[...]
    )
    h_ref[...] = h
    logits = jnp.dot(h, wg_ref[...], preferred_element_type=jnp.float32)
    gate, idx = _router(logits)
    g_ref[...] = gate
    i_ref[...] = idx


def _dense_router(x, w, b, wg, tile=256):
    wgp = jnp.pad(wg, ((0, 0), (0, EPAD - E)))
    return pl.pallas_call(
        _dense_router_body,
        grid=(T // tile,),
        in_specs=[
            pl.BlockSpec((tile, D), lambda t: (t, 0)),
            pl.BlockSpec((D, D), lambda t: (0, 0)),
            pl.BlockSpec((1, D), lambda t: (0, 0)),
            pl.BlockSpec((D, EPAD), lambda t: (0, 0)),
        ],
        out_specs=[
            pl.BlockSpec((tile, D), lambda t: (t, 0)),
            pl.BlockSpec((tile, 1), lambda t: (t, 0)),
            pl.BlockSpec((tile, 1), lambda t: (t, 0)),
        ],
        out_shape=[
            jax.ShapeDtypeStruct((T, D), jnp.float32),
            jax.ShapeDtypeStruct((T, 1), jnp.float32),
            jax.ShapeDtypeStruct((T, 1), jnp.int32),
        ],
    )(x, w, b.reshape(1, D), wgp)


def _splat(x):
    return jax.lax.broadcast_in_dim(x, (16,), ())


def _take(v, idx):
    return v.at[idx].get(mode="promise_in_bounds")


def _excl_cumsum(x, lane):
    zero = jnp.zeros((16,), jnp.int32)
    incl = x
    for k in (1, 2, 4, 8):
        shifted = _take(incl, jnp.maximum(lane - k, 0))
        incl = incl + jnp.where(lane >= jnp.full((16,), k, jnp.int32),
                                shifted, zero)
    return incl - x


def _offsets(histall, wid, lane, batch_split=None):
    tot = jnp.zeros((16,), jnp.int32)
    r0 = jnp.zeros((16,), jnp.int32)
    c0 = jnp.zeros((16,), jnp.int32)
    for w in range(NW):
        row = histall[w, :]
        tot = tot + row
        m = lax.convert_element_type(w < wid, jnp.int32)
        r0 = r0 + row * _splat(m)
        if batch_split is not None and w < batch_split:
            c0 = c0 + row
    cpad = ((tot + (TILE - 1)) >> TILE_SHIFT) << TILE_SHIFT
    opad = _excl_cumsum(cpad, lane)
    return tot, c0, opad, r0


def _tile_experts(opad, tebuf):
    for chunk in range(3):
        r = (lax.iota(jnp.int32, 16) + chunk * 16) * TILE
        te = jnp.zeros((16,), jnp.int32)
        for e in range(1, E):
            te = jnp.where(r >= _splat(opad[e]),
                           jnp.full((16,), e, jnp.int32), te)
        tebuf[pl.ds(chunk * 16, 16)] = te


def _lane_positions(evec, cnt, lane):
    posvec = jnp.zeros((16,), jnp.int32)
    one = jnp.ones((16,), jnp.int32)
    zero = jnp.zeros((16,), jnp.int32)
    for l in range(16):
        evl = _splat(evec[l])
        onehot = lane == evl
        pvec = _take(cnt, evl)
        cnt = cnt + jnp.where(onehot, one, zero)
        posvec = jnp.where(lane == jnp.full((16,), l, jnp.int32), pvec,
                           posvec)
    return posvec, cnt


def _sc_mesh():
    return plsc.VectorSubcoreMesh(
        core_axis_name="c", subcore_axis_name="s", num_cores=1
    )


def _hist_loop(buf, off, lane, hist):
    one = jnp.ones((16,), jnp.int32)
    zero = jnp.zeros((16,), jnp.int32)

    def h_body(g, h):
        ev = buf[pl.ds(off + g * 16, 16)]
        for l in range(16):
            h = h + jnp.where(lane == _splat(ev[l]), one, zero)
        return h

    return lax.fori_loop(0, HALF // 16, h_body, hist)


def _pos_loop(buf, off, posbuf, lane, cnt):
    def body(g, c):
        ev = buf[pl.ds(off + g * 16, 16)]
        posvec, c = _lane_positions(ev, c, lane)
        posbuf[pl.ds(g * 16, 16)] = posvec
        return c

    return lax.fori_loop(0, HALF // 16, body, cnt)


def _route2(hidden, idx2, gate2):
    @functools.partial(
        pl.kernel,
        mesh=_sc_mesh(),
        out_type=[
            jax.ShapeDtypeStruct((TP, D), jnp.float32),
            jax.ShapeDtypeStruct((TP,), jnp.float32),
            jax.ShapeDtypeStruct((T,), jnp.int32),
            jax.ShapeDtypeStruct((48,), jnp.int32),
            jax.ShapeDtypeStruct((NW, 16), jnp.int32),
        ],
        scratch_types=[
            pltpu.VMEM((CH,), jnp.int32),
            pltpu.VMEM((HALF, D), jnp.float32),
            pltpu.VMEM((HALF,), jnp.float32),
            pltpu.VMEM((HALF,), jnp.float32),
            pltpu.VMEM((HALF,), jnp.int32),
            pltpu.VMEM((HALF,), jnp.int32),
            pltpu.VMEM((16,), jnp.int32),
            pltpu.VMEM((NW, 16), jnp.int32),
            pltpu.VMEM((48,), jnp.int32),
            pltpu.SemaphoreType.DMA,
        ],
    )
    def k(hid_ref, idx_ref, gate_ref, xs_ref, gs_ref, pos_ref, te_ref,
          hx_ref, idxbuf, rowbuf, gbuf_a, gbuf_b, posbuf_a, posbuf_b,
          histv, histall, tebuf, sem):
        wid = lax.axis_index("s")
        base = wid * CH
        lane = lax.iota(jnp.int32, 16)
        zero = jnp.zeros((16,), jnp.int32)
        pltpu.sync_copy(idx_ref.at[pl.ds(base, CH)], idxbuf)
        pltpu.sync_copy(gate_ref.at[pl.ds(base, HALF)], gbuf_a)
        pltpu.sync_copy(gate_ref.at[pl.ds(base + HALF, HALF)], gbuf_b)

        hist = _hist_loop(idxbuf, 0, lane, zero)
        histv[...] = _hist_loop(idxbuf, HALF, lane, hist)
        pltpu.sync_copy(histv, hx_ref.at[wid])
        plsc.subcore_barrier()
        pltpu.sync_copy(hx_ref, histall)

        _, _, opad, r0 = _offsets(histall, wid, lane)
        cnt = _pos_loop(idxbuf, 0, posbuf_a, lane, opad + r0)
        _pos_loop(idxbuf, HALF, posbuf_b, lane, cnt)

        pltpu.sync_copy(hid_ref.at[pl.ds(base, HALF)], rowbuf)
        pltpu.async_copy(rowbuf, xs_ref.at[posbuf_a], sem).wait()
        pltpu.sync_copy(hid_ref.at[pl.ds(base + HALF, HALF)], rowbuf)
        pltpu.async_copy(rowbuf, xs_ref.at[posbuf_b], sem).wait()
        pltpu.async_copy(gbuf_a, gs_ref.at[posbuf_a], sem).wait()
        pltpu.async_copy(gbuf_b, gs_ref.at[posbuf_b], sem).wait()
        pltpu.sync_copy(posbuf_a, pos_ref.at[pl.ds(base, HALF)])
        pltpu.sync_copy(posbuf_b, pos_ref.at[pl.ds(base + HALF, HALF)])

        @pl.when(wid == 0)
        def _():
            _tile_experts(opad, tebuf)
            pltpu.sync_copy(tebuf, te_ref)

    xs, gs, pos2, te, _ = k(hidden, idx2, gate2)
    return xs, gs, pos2, te


def _gmm_router_body(te_ref, xs_ref, gs_ref, We_ref, be_ref, wg_ref,
                     ys_ref, g3_ref, i3_ref):
    yraw = (
        jnp.dot(xs_ref[...], We_ref[0], preferred_element_type=jnp.float32)
        + be_ref[0]
    )
    ys = gs_ref[...] * yraw
    ys_ref[...] = ys
    logits = jnp.dot(ys, wg_ref[...], preferred_element_type=jnp.float32)
    gate, idx = _router(logits)
    g3_ref[...] = gate
    i3_ref[...] = idx


def _gmm_router(te, xs, gs, We, be, wg):
    wgp = jnp.pad(wg, ((0, 0), (0, EPAD - E)))
    return pl.pallas_call(
        _gmm_router_body,
        grid_spec=pltpu.PrefetchScalarGridSpec(
            num_scalar_prefetch=1,
            grid=(XT,),
            in_specs=[
                pl.BlockSpec((TILE, D), lambda t, te: (t, 0)),
                pl.BlockSpec((TILE, 1), lambda t, te: (t, 0)),
                pl.BlockSpec((1, D, D), lambda t, te: (te[t], 0, 0)),
                pl.BlockSpec((1, 1, D), lambda t, te: (te[t], 0, 0)),
                pl.BlockSpec((D, EPAD), lambda t, te: (0, 0)),
            ],
            out_specs=[
                pl.BlockSpec((TILE, D), lambda t, te: (t, 0)),
                pl.BlockSpec((TILE, 1), lambda t, te: (t, 0)),
                pl.BlockSpec((TILE, 1), lambda t, te: (t, 0)),
            ],
        ),
        out_shape=[
            jax.ShapeDtypeStruct((TP, D), jnp.float32),
            jax.ShapeDtypeStruct((TP, 1), jnp.float32),
            jax.ShapeDtypeStruct((TP, 1), jnp.int32),
        ],
    )(te, xs, gs.reshape(TP, 1), We, be.reshape(E, 1, D), wgp)


def _route3(ys, i3, g3, pos2):
    @functools.partial(
        pl.kernel,
        mesh=_sc_mesh(),
        out_type=[
            jax.ShapeDtypeStruct((TP, D), jnp.float32),
            jax.ShapeDtypeStruct((TP,), jnp.float32),
            jax.ShapeDtypeStruct((48,), jnp.int32),
            jax.ShapeDtypeStruct((48,), jnp.int32),
            jax.ShapeDtypeStruct((NW, 16), jnp.int32),
        ],
        scratch_types=[
            pltpu.VMEM((CH,), jnp.int32),
            pltpu.VMEM((HALF,), jnp.int32),
            pltpu.VMEM((HALF,), jnp.int32),
            pltpu.VMEM((HALF,), jnp.int32),
            pltpu.VMEM((HALF,), jnp.int32),
            pltpu.VMEM((HALF,), jnp.float32),
            pltpu.VMEM((HALF,), jnp.float32),
            pltpu.VMEM((HALF, D), jnp.float32),
            pltpu.VMEM((HALF,), jnp.int32),
            pltpu.VMEM((HALF,), jnp.int32),
            pltpu.VMEM((16,), jnp.int32),
            pltpu.VMEM((NW, 16), jnp.int32),
            pltpu.VMEM((48,), jnp.int32),
            pltpu.VMEM((48,), jnp.int32),
            pltpu.SemaphoreType.DMA,
        ],
    )
    def k(ys_ref, i3_ref, g3_ref, pos2_ref, xs3_ref, gs3_ref, te_ref,
          bnd_ref, hx_ref, pbuf, pbuf_a, pbuf_b, ivals_a, ivals_b,
          gvals_a, gvals_b, rowbuf, posbuf_a, posbuf_b, histv, histall,
          tebuf, bndbuf, sem):
        wid = lax.axis_index("s")
        base = wid * CH
        lane = lax.iota(jnp.int32, 16)
        zero = jnp.zeros((16,), jnp.int32)
        pltpu.sync_copy(pos2_ref.at[pl.ds(base, CH)], pbuf)
        for kk in range(HALF // 16):
            pbuf_a[pl.ds(kk * 16, 16)] = pbuf[pl.ds(kk * 16, 16)]
            pbuf_b[pl.ds(kk * 16, 16)] = pbuf[pl.ds(HALF + kk * 16, 16)]
        pltpu.async_copy(i3_ref.at[pbuf_a], ivals_a, sem).wait()
        pltpu.async_copy(i3_ref.at[pbuf_b], ivals_b, sem).wait()
        pltpu.async_copy(g3_ref.at[pbuf_a], gvals_a, sem).wait()
        pltpu.async_copy(g3_ref.at[pbuf_b], gvals_b, sem).wait()

        hist = _hist_loop(ivals_a, 0, lane, zero)
        histv[...] = _hist_loop(ivals_b, 0, lane, hist)
        pltpu.sync_copy(histv, hx_ref.at[wid])
        plsc.subcore_barrier()
        pltpu.sync_copy(hx_ref, histall)

        tot, c0, opad, r0 = _offsets(histall, wid, lane,
                                     batch_split=NW // 2)
        cnt = _pos_loop(ivals_a, 0, posbuf_a, lane, opad + r0)
        _pos_loop(ivals_b, 0, posbuf_b, lane, cnt)

        pltpu.async_copy(ys_ref.at[pbuf_a], rowbuf, sem).wait()
        pltpu.async_copy(rowbuf, xs3_ref.at[posbuf_a], sem).wait()
        pltpu.async_copy(ys_ref.at[pbuf_b], rowbuf, sem).wait()
        pltpu.async_copy(rowbuf, xs3_ref.at[posbuf_b], sem).wait()
        pltpu.async_copy(gvals_a, gs3_ref.at[posbuf_a], sem).wait()
        pltpu.async_copy(gvals_b, gs3_ref.at[posbuf_b], sem).wait()

        @pl.when(wid == 0)
        def _():
            _tile_experts(opad, tebuf)
            pltpu.sync_copy(tebuf, te_ref)
            bndbuf[pl.ds(0, 16)] = opad
            bndbuf[pl.ds(16, 16)] = c0
            bndbuf[pl.ds(32, 16)] = tot
            pltpu.sync_copy(bndbuf, bnd_ref)

    xs3, gs3, te3, bnd, _ = k(ys, i3, g3, pos2)
    return xs3, gs3, te3, bnd


def _gmm_body(te_ref, xs_ref, gs_ref, We_ref, be_ref, ys_ref):
    yraw = (
        jnp.dot(xs_ref[...], We_ref[0], preferred_element_type=jnp.float32)
        + be_ref[0]
    )
    ys_ref[...] = gs_ref[...] * yraw


def _gmm(te, xs, gs, We, be):
    return pl.pallas_call(
        _gmm_body,
        grid_spec=pltpu.PrefetchScalarGridSpec(
            num_scalar_prefetch=1,
            grid=(XT,),
            in_specs=[
                pl.BlockSpec((TILE, D), lambda t, te: (t, 0)),
                pl.BlockSpec((TILE, 1), lambda t, te: (t, 0)),
                pl.BlockSpec((1, D, D), lambda t, te: (te[t], 0, 0)),
                pl.BlockSpec((1, 1, D), lambda t, te: (te[t], 0, 0)),
            ],
            out_specs=pl.BlockSpec((TILE, D), lambda t, te: (t, 0)),
        ),
        out_shape=jax.ShapeDtypeStruct((TP, D), jnp.float32),
    )(te, xs, gs.reshape(TP, 1), We, be.reshape(E, 1, D))


def _final_body(h_ref, y3_ref, bnd_ref, y_ref, loss_ref):
    h = h_ref[...]
    hs0 = jnp.sum(h[:S], axis=0, keepdims=True)
    hs1 = jnp.sum(h[S:], axis=0, keepdims=True)
    r = jax.lax.broadcasted_iota(jnp.int32, (TP, 1), 0)
    m0 = jnp.zeros((TP, 1), jnp.bool_)
    m1 = jnp.zeros((TP, 1), jnp.bool_)
    for e in range(E):
        off = bnd_ref[e]
        c0 = bnd_ref[16 + e]
        c = bnd_ref[32 + e]
        m0 = m0 | ((r >= off) & (r < off + c0))
        m1 = m1 | ((r >= off + c0) & (r < off + c))
    y3 = y3_ref[...]
    zero = jnp.zeros_like(y3)
    s0 = jnp.sum(jnp.where(m0, y3, zero), axis=0, keepdims=True)
    s1 = jnp.sum(jnp.where(m1, y3, zero), axis=0, keepdims=True)
    emb = jnp.concatenate([hs0 + s0, hs1 + s1], axis=0) / S
    m = jnp.max(emb, axis=1, keepdims=True)
    lse = m + jnp.log(jnp.sum(jnp.exp(emb - m), axis=1, keepdims=True))
    logp = emb - lse
    col = jax.lax.broadcasted_iota(jnp.int32, logp.shape, 1)
    v0 = jnp.sum(jnp.where(col[0:1] == y_ref[0], logp[0:1], 0.0))
    v1 = jnp.sum(jnp.where(col[1:2] == y_ref[1], logp[1:2], 0.0))
    loss_ref[0, 0] = -(v0 + v1) / B


def _final(hidden, ys3, bnd, y):
    out = pl.pallas_call(
        _final_body,
        in_specs=[
            pl.BlockSpec((T, D), lambda: (0, 0)),
            pl.BlockSpec((TP, D), lambda: (0, 0)),
            pl.BlockSpec(memory_space=pltpu.SMEM),
            pl.BlockSpec(memory_space=pltpu.SMEM),
        ],
        out_specs=pl.BlockSpec(memory_space=pltpu.SMEM),
        out_shape=jax.ShapeDtypeStruct((1, 1), jnp.float32),
    )(hidden, ys3, bnd, y)
    return out.reshape(())


def kernel(x, y, W1, b1, wg2, We2, be2, wg3, We3, be3):
    xt = x.reshape(T, D)
    hidden, g2, i2 = _dense_router(xt, W1.T, b1, wg2)
    xs, gs, pos2, te = _route2(hidden, i2.reshape(T), g2.reshape(T))
    ys, g3, i3 = _gmm_router(te, xs, gs, We2, be2, wg3)
    xs3, gs3, te3, bnd = _route3(ys, i3.reshape(TP), g3.reshape(TP), pos2)
    ys3 = _gmm(te3, xs3, gs3, We3, be3)
    return _final(hidden, ys3, bnd, y.astype(jnp.int32))


# speedup vs baseline: 1.3913x; 1.0232x over previous
import functools

import jax
import jax.numpy as jnp
from jax import lax
from jax.experimental import pallas as pl
from jax.experimental.pallas import tpu as pltpu
from jax.experimental.pallas import tpu_sc as plsc

B, S, D, E = 2, 2048, 768, 8
T = B * S
EPAD = 128
TILE = 128
TILE_SHIFT = 7
TP = T + E * TILE
XT = TP // TILE
NW = 16
CH = T // NW
HALF = CH // 2


def _router(logits):
    col = jax.lax.broadcasted_iota(jnp.int32, logits.shape, 1)
    valid = col < E
    logits = jnp.where(valid, logits, -jnp.inf)
    m = jnp.max(logits, axis=1, keepdims=True)
    p = jnp.exp(logits - m)
    s = jnp.sum(p, axis=1, keepdims=True)
    gate = jnp.max(p, axis=1, keepdims=True) / s
    is_max = (logits == m) & valid
    idx = jnp.min(jnp.where(is_max, col, EPAD), axis=1, keepdims=True)
    return gate, idx


def _dense_router_body(x_ref, w_ref, b_ref, wg_ref, h_ref, g_ref, i_ref):
    h = (
        jnp.dot(x_ref[...], w_ref[...], preferred_element_type=jnp.float32)
        + b_ref[...]
    )
    h_ref[...] = h
    logits = jnp.dot(h, wg_ref[...], preferred_element_type=jnp.float32)
    gate, idx = _router(logits)
    g_ref[...] = gate
    i_ref[...] = idx


def _dense_router(x, w, b, wg, tile=256):
    wgp = jnp.pad(wg, ((0, 0), (0, EPAD - E)))
    return pl.pallas_call(
        _dense_router_body,
        grid=(T // tile,),
        in_specs=[
            pl.BlockSpec((tile, D), lambda t: (t, 0)),
            pl.BlockSpec((D, D), lambda t: (0, 0)),
            pl.BlockSpec((1, D), lambda t: (0, 0)),
            pl.BlockSpec((D, EPAD), lambda t: (0, 0)),
        ],
        out_specs=[
            pl.BlockSpec((tile, D), lambda t: (t, 0)),
            pl.BlockSpec((tile, 1), lambda t: (t, 0)),
            pl.BlockSpec((tile, 1), lambda t: (t, 0)),
        ],
        out_shape=[
            jax.ShapeDtypeStruct((T, D), jnp.float32),
            jax.ShapeDtypeStruct((T, 1), jnp.float32),
            jax.ShapeDtypeStruct((T, 1), jnp.int32),
        ],
    )(x, w, b.reshape(1, D), wgp)


def _splat(x):
    return jax.lax.broadcast_in_dim(x, (16,), ())


def _take(v, idx):
    return v.at[idx].get(mode="promise_in_bounds")


def _excl_cumsum(x, lane):
    zero = jnp.zeros((16,), jnp.int32)
    incl = x
    for k in (1, 2, 4, 8):
        shifted = _take(incl, jnp.maximum(lane - k, 0))
        incl = incl + jnp.where(lane >= jnp.full((16,), k, jnp.int32),
                                shifted, zero)
    return incl - x


def _offsets(histall, wid, lane, batch_split=None):
    tot = jnp.zeros((16,), jnp.int32)
    r0 = jnp.zeros((16,), jnp.int32)
    c0 = jnp.zeros((16,), jnp.int32)
    for w in range(NW):
        row = histall[w, :]
        tot = tot + row
        m = lax.convert_element_type(w < wid, jnp.int32)
        r0 = r0 + row * _splat(m)
        if batch_split is not None and w < batch_split:
            c0 = c0 + row
    cpad = ((tot + (TILE - 1)) >> TILE_SHIFT) << TILE_SHIFT
    opad = _excl_cumsum(cpad, lane)
    return tot, c0, opad, r0


def _tile_experts(opad, tebuf):
    for chunk in range(3):
        r = (lax.iota(jnp.int32, 16) + chunk * 16) * TILE
        te = jnp.zeros((16,), jnp.int32)
        for e in range(1, E):
            te = jnp.where(r >= _splat(opad[e]),
                           jnp.full((16,), e, jnp.int32), te)
        tebuf[pl.ds(chunk * 16, 16)] = te


def _lane_positions(evec, cnt, lane):
    posvec = jnp.zeros((16,), jnp.int32)
    one = jnp.ones((16,), jnp.int32)
    zero = jnp.zeros((16,), jnp.int32)
    for l in range(16):
        evl = _splat(evec[l])
        onehot = lane == evl
        pvec = _take(cnt, evl)
        cnt = cnt + jnp.where(onehot, one, zero)
        posvec = jnp.where(lane == jnp.full((16,), l, jnp.int32), pvec,
                           posvec)
    return posvec, cnt


def _sc_mesh():
    return plsc.VectorSubcoreMesh(
        core_axis_name="c", subcore_axis_name="s", num_cores=1
    )


def _hist_loop(buf, off, lane, hist):
    one = jnp.ones((16,), jnp.int32)
    zero = jnp.zeros((16,), jnp.int32)

    def h_body(g, h):
        ev = buf[pl.ds(off + g * 16, 16)]
        for l in range(16):
            h = h + jnp.where(lane == _splat(ev[l]), one, zero)
        return h

    return lax.fori_loop(0, HALF // 16, h_body, hist)


def _pos_loop(buf, off, posbuf, lane, cnt):
    def body(g, c):
        ev = buf[pl.ds(off + g * 16, 16)]
        posvec, c = _lane_positions(ev, c, lane)
        posbuf[pl.ds(g * 16, 16)] = posvec
        return c

    return lax.fori_loop(0, HALF // 16, body, cnt)


def _route2(hidden, idx2, gate2):
    @functools.partial(
        pl.kernel,
        mesh=_sc_mesh(),
        out_type=[
            jax.ShapeDtypeStruct((TP, D), jnp.float32),
            jax.ShapeDtypeStruct((TP,), jnp.float32),
            jax.ShapeDtypeStruct((T,), jnp.int32),
            jax.ShapeDtypeStruct((48,), jnp.int32),
            jax.ShapeDtypeStruct((NW, 16), jnp.int32),
        ],
        scratch_types=[
            pltpu.VMEM((CH,), jnp.int32),
            pltpu.VMEM((HALF, D), jnp.float32),
            pltpu.VMEM((HALF,), jnp.float32),
            pltpu.VMEM((HALF,), jnp.float32),
            pltpu.VMEM((HALF,), jnp.int32),
            pltpu.VMEM((HALF,), jnp.int32),
            pltpu.VMEM((16,), jnp.int32),
            pltpu.VMEM((NW, 16), jnp.int32),
            pltpu.VMEM((48,), jnp.int32),
            pltpu.SemaphoreType.DMA,
        ],
    )
    def k(hid_ref, idx_ref, gate_ref, xs_ref, gs_ref, pos_ref, te_ref,
          hx_ref, idxbuf, rowbuf, gbuf_a, gbuf_b, posbuf_a, posbuf_b,
          histv, histall, tebuf, sem):
        wid = lax.axis_index("s")
        base = wid * CH
        lane = lax.iota(jnp.int32, 16)
        zero = jnp.zeros((16,), jnp.int32)
        pltpu.sync_copy(idx_ref.at[pl.ds(base, CH)], idxbuf)
        pltpu.sync_copy(gate_ref.at[pl.ds(base, HALF)], gbuf_a)
        pltpu.sync_copy(gate_ref.at[pl.ds(base + HALF, HALF)], gbuf_b)

        hist = _hist_loop(idxbuf, 0, lane, zero)
        histv[...] = _hist_loop(idxbuf, HALF, lane, hist)
        pltpu.sync_copy(histv, hx_ref.at[wid])
        plsc.subcore_barrier()
        pltpu.sync_copy(hx_ref, histall)

        _, _, opad, r0 = _offsets(histall, wid, lane)
        cnt = _pos_loop(idxbuf, 0, posbuf_a, lane, opad + r0)
        _pos_loop(idxbuf, HALF, posbuf_b, lane, cnt)

        pltpu.sync_copy(hid_ref.at[pl.ds(base, HALF)], rowbuf)
        pltpu.async_copy(rowbuf, xs_ref.at[posbuf_a], sem).wait()
        pltpu.sync_copy(hid_ref.at[pl.ds(base + HALF, HALF)], rowbuf)
        pltpu.async_copy(rowbuf, xs_ref.at[posbuf_b], sem).wait()
        pltpu.async_copy(gbuf_a, gs_ref.at[posbuf_a], sem).wait()
        pltpu.async_copy(gbuf_b, gs_ref.at[posbuf_b], sem).wait()
        pltpu.sync_copy(posbuf_a, pos_ref.at[pl.ds(base, HALF)])
        pltpu.sync_copy(posbuf_b, pos_ref.at[pl.ds(base + HALF, HALF)])

        @pl.when(wid == 0)
        def _():
            _tile_experts(opad, tebuf)
            pltpu.sync_copy(tebuf, te_ref)

    xs, gs, pos2, te, _ = k(hidden, idx2, gate2)
    return xs, gs, pos2, te


def _gmm_router_body(te_ref, xs_ref, gs_ref, We_ref, be_ref, wg_ref,
                     ys_ref, g3_ref, i3_ref):
    yraw = (
        jnp.dot(xs_ref[...].astype(jnp.bfloat16), We_ref[0],
                preferred_element_type=jnp.float32)
        + be_ref[0]
    )
    ys = gs_ref[...] * yraw
    ys_ref[...] = ys
    logits = jnp.dot(ys, wg_ref[...], preferred_element_type=jnp.float32)
    gate, idx = _router(logits)
    g3_ref[...] = gate
    i3_ref[...] = idx


def _gmm_router(te, xs, gs, We, be, wg):
    wgp = jnp.pad(wg, ((0, 0), (0, EPAD - E)))
    return pl.pallas_call(
        _gmm_router_body,
        grid_spec=pltpu.PrefetchScalarGridSpec(
            num_scalar_prefetch=1,
            grid=(XT,),
            in_specs=[
                pl.BlockSpec((TILE, D), lambda t, te: (t, 0)),
                pl.BlockSpec((TILE, 1), lambda t, te: (t, 0)),
                pl.BlockSpec((1, D, D), lambda t, te: (te[t], 0, 0)),
                pl.BlockSpec((1, 1, D), lambda t, te: (te[t], 0, 0)),
                pl.BlockSpec((D, EPAD), lambda t, te: (0, 0)),
            ],
            out_specs=[
                pl.BlockSpec((TILE, D), lambda t, te: (t, 0)),
                pl.BlockSpec((TILE, 1), lambda t, te: (t, 0)),
                pl.BlockSpec((TILE, 1), lambda t, te: (t, 0)),
            ],
        ),
        out_shape=[
            jax.ShapeDtypeStruct((TP, D), jnp.float32),
            jax.ShapeDtypeStruct((TP, 1), jnp.float32),
            jax.ShapeDtypeStruct((TP, 1), jnp.int32),
        ],
    )(te, xs, gs.reshape(TP, 1), We.astype(jnp.bfloat16),
      be.reshape(E, 1, D), wgp)


def _route3(ys, i3, g3, pos2):
    @functools.partial(
        pl.kernel,
        mesh=_sc_mesh(),
        out_type=[
            jax.ShapeDtypeStruct((TP, D), jnp.float32),
            jax.ShapeDtypeStruct((TP,), jnp.float32),
            jax.ShapeDtypeStruct((48,), jnp.int32),
            jax.ShapeDtypeStruct((48,), jnp.int32),
            jax.ShapeDtypeStruct((NW, 16), jnp.int32),
        ],
        scratch_types=[
            pltpu.VMEM((CH,), jnp.int32),
            pltpu.VMEM((HALF,), jnp.int32),
            pltpu.VMEM((HALF,), jnp.int32),
            pltpu.VMEM((HALF,), jnp.int32),
            pltpu.VMEM((HALF,), jnp.int32),
            pltpu.VMEM((HALF,), jnp.float32),
            pltpu.VMEM((HALF,), jnp.float32),
            pltpu.VMEM((HALF, D), jnp.float32),
            pltpu.VMEM((HALF,), jnp.int32),
            pltpu.VMEM((HALF,), jnp.int32),
            pltpu.VMEM((16,), jnp.int32),
            pltpu.VMEM((NW, 16), jnp.int32),
            pltpu.VMEM((48,), jnp.int32),
            pltpu.VMEM((48,), jnp.int32),
            pltpu.SemaphoreType.DMA,
        ],
    )
    def k(ys_ref, i3_ref, g3_ref, pos2_ref, xs3_ref, gs3_ref, te_ref,
          bnd_ref, hx_ref, pbuf, pbuf_a, pbuf_b, ivals_a, ivals_b,
          gvals_a, gvals_b, rowbuf, posbuf_a, posbuf_b, histv, histall,
          tebuf, bndbuf, sem):
        wid = lax.axis_index("s")
        base = wid * CH
        lane = lax.iota(jnp.int32, 16)
        zero = jnp.zeros((16,), jnp.int32)
        pltpu.sync_copy(pos2_ref.at[pl.ds(base, CH)], pbuf)
        for kk in range(HALF // 16):
            pbuf_a[pl.ds(kk * 16, 16)] = pbuf[pl.ds(kk * 16, 16)]
            pbuf_b[pl.ds(kk * 16, 16)] = pbuf[pl.ds(HALF + kk * 16, 16)]
        pltpu.async_copy(i3_ref.at[pbuf_a], ivals_a, sem).wait()
        pltpu.async_copy(i3_ref.at[pbuf_b], ivals_b, sem).wait()
        pltpu.async_copy(g3_ref.at[pbuf_a], gvals_a, sem).wait()
        pltpu.async_copy(g3_ref.at[pbuf_b], gvals_b, sem).wait()

        hist = _hist_loop(ivals_a, 0, lane, zero)
        histv[...] = _hist_loop(ivals_b, 0, lane, hist)
        pltpu.sync_copy(histv, hx_ref.at[wid])
        plsc.subcore_barrier()
        pltpu.sync_copy(hx_ref, histall)

        tot, c0, opad, r0 = _offsets(histall, wid, lane,
                                     batch_split=NW // 2)
        cnt = _pos_loop(ivals_a, 0, posbuf_a, lane, opad + r0)
        _pos_loop(ivals_b, 0, posbuf_b, lane, cnt)

        pltpu.async_copy(ys_ref.at[pbuf_a], rowbuf, sem).wait()
        pltpu.async_copy(rowbuf, xs3_ref.at[posbuf_a], sem).wait()
        pltpu.async_copy(ys_ref.at[pbuf_b], rowbuf, sem).wait()
        pltpu.async_copy(rowbuf, xs3_ref.at[posbuf_b], sem).wait()
        pltpu.async_copy(gvals_a, gs3_ref.at[posbuf_a], sem).wait()
        pltpu.async_copy(gvals_b, gs3_ref.at[posbuf_b], sem).wait()

        @pl.when(wid == 0)
        def _():
            _tile_experts(opad, tebuf)
            pltpu.sync_copy(tebuf, te_ref)
            bndbuf[pl.ds(0, 16)] = opad
            bndbuf[pl.ds(16, 16)] = c0
            bndbuf[pl.ds(32, 16)] = tot
            pltpu.sync_copy(bndbuf, bnd_ref)

    xs3, gs3, te3, bnd, _ = k(ys, i3, g3, pos2)
    return xs3, gs3, te3, bnd


def _gmm_body(te_ref, xs_ref, gs_ref, We_ref, be_ref, ys_ref):
    yraw = (
        jnp.dot(xs_ref[...].astype(jnp.bfloat16), We_ref[0],
                preferred_element_type=jnp.float32)
        + be_ref[0]
    )
    ys_ref[...] = gs_ref[...] * yraw


def _gmm(te, xs, gs, We, be):
    return pl.pallas_call(
        _gmm_body,
        grid_spec=pltpu.PrefetchScalarGridSpec(
            num_scalar_prefetch=1,
            grid=(XT,),
            in_specs=[
                pl.BlockSpec((TILE, D), lambda t, te: (t, 0)),
                pl.BlockSpec((TILE, 1), lambda t, te: (t, 0)),
                pl.BlockSpec((1, D, D), lambda t, te: (te[t], 0, 0)),
                pl.BlockSpec((1, 1, D), lambda t, te: (te[t], 0, 0)),
            ],
            out_specs=pl.BlockSpec((TILE, D), lambda t, te: (t, 0)),
        ),
        out_shape=jax.ShapeDtypeStruct((TP, D), jnp.float32),
    )(te, xs, gs.reshape(TP, 1), We.astype(jnp.bfloat16),
      be.reshape(E, 1, D))


def _final_body(h_ref, y3_ref, bnd_ref, y_ref, loss_ref):
    h = h_ref[...]
    hs0 = jnp.sum(h[:S], axis=0, keepdims=True)
    hs1 = jnp.sum(h[S:], axis=0, keepdims=True)
    r = jax.lax.broadcasted_iota(jnp.int32, (TP, 1), 0)
    m0 = jnp.zeros((TP, 1), jnp.bool_)
    m1 = jnp.zeros((TP, 1), jnp.bool_)
    for e in range(E):
        off = bnd_ref[e]
        c0 = bnd_ref[16 + e]
        c = bnd_ref[32 + e]
        m0 = m0 | ((r >= off) & (r < off + c0))
        m1 = m1 | ((r >= off + c0) & (r < off + c))
    y3 = y3_ref[...]
    zero = jnp.zeros_like(y3)
    s0 = jnp.sum(jnp.where(m0, y3, zero), axis=0, keepdims=True)
    s1 = jnp.sum(jnp.where(m1, y3, zero), axis=0, keepdims=True)
    emb = jnp.concatenate([hs0 + s0, hs1 + s1], axis=0) / S
    m = jnp.max(emb, axis=1, keepdims=True)
    lse = m + jnp.log(jnp.sum(jnp.exp(emb - m), axis=1, keepdims=True))
    logp = emb - lse
    col = jax.lax.broadcasted_iota(jnp.int32, logp.shape, 1)
    v0 = jnp.sum(jnp.where(col[0:1] == y_ref[0], logp[0:1], 0.0))
    v1 = jnp.sum(jnp.where(col[1:2] == y_ref[1], logp[1:2], 0.0))
    loss_ref[0, 0] = -(v0 + v1) / B


def _final(hidden, ys3, bnd, y):
    out = pl.pallas_call(
        _final_body,
        in_specs=[
            pl.BlockSpec((T, D), lambda: (0, 0)),
            pl.BlockSpec((TP, D), lambda: (0, 0)),
            pl.BlockSpec(memory_space=pltpu.SMEM),
            pl.BlockSpec(memory_space=pltpu.SMEM),
        ],
        out_specs=pl.BlockSpec(memory_space=pltpu.SMEM),
        out_shape=jax.ShapeDtypeStruct((1, 1), jnp.float32),
    )(hidden, ys3, bnd, y)
    return out.reshape(())


def kernel(x, y, W1, b1, wg2, We2, be2, wg3, We3, be3):
    xt = x.reshape(T, D)
    hidden, g2, i2 = _dense_router(xt.astype(jnp.bfloat16),
                                   W1.T.astype(jnp.bfloat16), b1, wg2)
    xs, gs, pos2, te = _route2(hidden, i2.reshape(T), g2.reshape(T))
    ys, g3, i3 = _gmm_router(te, xs, gs, We2, be2, wg3)
    xs3, gs3, te3, bnd = _route3(ys, i3.reshape(TP), g3.reshape(TP), pos2)
    ys3 = _gmm(te3, xs3, gs3, We3, be3)
    return _final(hidden, ys3, bnd, y.astype(jnp.int32))


# speedup vs baseline: 1.5086x; 1.0843x over previous
import functools

import jax
import jax.numpy as jnp
from jax import lax
from jax.experimental import pallas as pl
from jax.experimental.pallas import tpu as pltpu
from jax.experimental.pallas import tpu_sc as plsc

B, S, D, E = 2, 2048, 768, 8
T = B * S
EPAD = 128
TILE = 128
TILE_SHIFT = 7
TP = T + E * TILE
XT = TP // TILE
XTP = 48
NW = 16
CH = T // NW
HALF = CH // 2


def _router(logits):
    col = jax.lax.broadcasted_iota(jnp.int32, logits.shape, 1)
    valid = col < E
    logits = jnp.where(valid, logits, -jnp.inf)
    m = jnp.max(logits, axis=1, keepdims=True)
    p = jnp.exp(logits - m)
    s = jnp.sum(p, axis=1, keepdims=True)
    gate = jnp.max(p, axis=1, keepdims=True) / s
    is_max = (logits == m) & valid
    idx = jnp.min(jnp.where(is_max, col, EPAD), axis=1, keepdims=True)
    return gate, idx


def _dense_router_body(x_ref, w_ref, b_ref, wg_ref, h_ref, g_ref, i_ref,
                       hs_ref):
    t = pl.program_id(0)
    h = (
        jnp.dot(x_ref[...], w_ref[...], preferred_element_type=jnp.float32)
        + b_ref[...]
    )
    h_ref[...] = h
    logits = jnp.dot(h, wg_ref[...], preferred_element_type=jnp.float32)
    gate, idx = _router(logits)
    g_ref[...] = gate
    i_ref[...] = idx
    tiles_per_batch = S // h.shape[0]
    colsum = jnp.sum(h, axis=0, keepdims=True)[None]

    @pl.when(t % tiles_per_batch == 0)
    def _():
        hs_ref[...] = jnp.zeros_like(hs_ref)

    hs_ref[...] += colsum


def _dense_router(x, w, b, wg, tile=256):
    wgp = jnp.pad(wg, ((0, 0), (0, EPAD - E)))
    return pl.pallas_call(
        _dense_router_body,
        grid=(T // tile,),
        in_specs=[
            pl.BlockSpec((tile, D), lambda t: (t, 0)),
            pl.BlockSpec((D, D), lambda t: (0, 0)),
            pl.BlockSpec((1, D), lambda t: (0, 0)),
            pl.BlockSpec((D, EPAD), lambda t: (0, 0)),
        ],
        out_specs=[
            pl.BlockSpec((tile, D), lambda t: (t, 0)),
            pl.BlockSpec((tile, 1), lambda t: (t, 0)),
            pl.BlockSpec((tile, 1), lambda t: (t, 0)),
            pl.BlockSpec((1, 1, D), lambda t: (t // (S // tile), 0, 0)),
        ],
        out_shape=[
            jax.ShapeDtypeStruct((T, D), jnp.float32),
            jax.ShapeDtypeStruct((T, 1), jnp.float32),
            jax.ShapeDtypeStruct((T, 1), jnp.int32),
            jax.ShapeDtypeStruct((B, 1, D), jnp.float32),
        ],
    )(x, w, b.reshape(1, D), wgp)


def _splat(x):
    return jax.lax.broadcast_in_dim(x, (16,), ())


def _take(v, idx):
    return v.at[idx].get(mode="promise_in_bounds")


def _excl_cumsum(x, lane):
    zero = jnp.zeros((16,), jnp.int32)
    incl = x
    for k in (1, 2, 4, 8):
        shifted = _take(incl, jnp.maximum(lane - k, 0))
        incl = incl + jnp.where(lane >= jnp.full((16,), k, jnp.int32),
                                shifted, zero)
    return incl - x


def _offsets(histall, wid, lane, batch_split=None):
    tot = jnp.zeros((16,), jnp.int32)
    r0 = jnp.zeros((16,), jnp.int32)
    c0 = jnp.zeros((16,), jnp.int32)
    for w in range(NW):
        row = histall[w, :]
        tot = tot + row
        m = lax.convert_element_type(w < wid, jnp.int32)
        r0 = r0 + row * _splat(m)
        if batch_split is not None and w < batch_split:
            c0 = c0 + row
    cpad = ((tot + (TILE - 1)) >> TILE_SHIFT) << TILE_SHIFT
    opad = _excl_cumsum(cpad, lane)
    return tot, c0, opad, r0


def _tile_experts(opad, tebuf):
    for chunk in range(3):
        r = (lax.iota(jnp.int32, 16) + chunk * 16) * TILE
        te = jnp.zeros((16,), jnp.int32)
        for e in range(1, E):
            te = jnp.where(r >= _splat(opad[e]),
                           jnp.full((16,), e, jnp.int32), te)
        tebuf[pl.ds(chunk * 16, 16)] = te


def _lane_positions(evec, cnt, lane):
    posvec = jnp.zeros((16,), jnp.int32)
    one = jnp.ones((16,), jnp.int32)
    zero = jnp.zeros((16,), jnp.int32)
    for l in range(16):
        evl = _splat(evec[l])
        onehot = lane == evl
        pvec = _take(cnt, evl)
        cnt = cnt + jnp.where(onehot, one, zero)
        posvec = jnp.where(lane == jnp.full((16,), l, jnp.int32), pvec,
                           posvec)
    return posvec, cnt


def _sc_mesh():
    return plsc.VectorSubcoreMesh(
        core_axis_name="c", subcore_axis_name="s", num_cores=1
    )


def _hist_loop(buf, off, lane, hist):
    one = jnp.ones((16,), jnp.int32)
    zero = jnp.zeros((16,), jnp.int32)

    def h_body(g, h):
        ev = buf[pl.ds(off + g * 16, 16)]
        for l in range(16):
            h = h + jnp.where(lane == _splat(ev[l]), one, zero)
        return h

    return lax.fori_loop(0, HALF // 16, h_body, hist)


def _pos_loop(buf, off, posbuf, lane, cnt):
    def body(g, c):
        ev = buf[pl.ds(off + g * 16, 16)]
        posvec, c = _lane_positions(ev, c, lane)
        posbuf[pl.ds(g * 16, 16)] = posvec
        return c

    return lax.fori_loop(0, HALF // 16, body, cnt)


def _route2(hidden, idx2, gate2):
    @functools.partial(
        pl.kernel,
        mesh=_sc_mesh(),
        out_type=[
            jax.ShapeDtypeStruct((TP, D), jnp.float32),
            jax.ShapeDtypeStruct((TP,), jnp.float32),
            jax.ShapeDtypeStruct((T,), jnp.int32),
            jax.ShapeDtypeStruct((48,), jnp.int32),
            jax.ShapeDtypeStruct((NW, 16), jnp.int32),
        ],
        scratch_types=[
            pltpu.VMEM((CH,), jnp.int32),
            pltpu.VMEM((HALF, D), jnp.float32),
            pltpu.VMEM((HALF,), jnp.float32),
            pltpu.VMEM((HALF,), jnp.float32),
            pltpu.VMEM((HALF,), jnp.int32),
            pltpu.VMEM((HALF,), jnp.int32),
            pltpu.VMEM((16,), jnp.int32),
            pltpu.VMEM((NW, 16), jnp.int32),
            pltpu.VMEM((48,), jnp.int32),
            pltpu.SemaphoreType.DMA,
        ],
    )
    def k(hid_ref, idx_ref, gate_ref, xs_ref, gs_ref, pos_ref, te_ref,
          hx_ref, idxbuf, rowbuf, gbuf_a, gbuf_b, posbuf_a, posbuf_b,
          histv, histall, tebuf, sem):
        wid = lax.axis_index("s")
        base = wid * CH
        lane = lax.iota(jnp.int32, 16)
        zero = jnp.zeros((16,), jnp.int32)
        pltpu.sync_copy(idx_ref.at[pl.ds(base, CH)], idxbuf)
        pltpu.sync_copy(gate_ref.at[pl.ds(base, HALF)], gbuf_a)
        pltpu.sync_copy(gate_ref.at[pl.ds(base + HALF, HALF)], gbuf_b)

        hist = _hist_loop(idxbuf, 0, lane, zero)
        histv[...] = _hist_loop(idxbuf, HALF, lane, hist)
        pltpu.sync_copy(histv, hx_ref.at[wid])
        plsc.subcore_barrier()
        pltpu.sync_copy(hx_ref, histall)

        _, _, opad, r0 = _offsets(histall, wid, lane)
        cnt = _pos_loop(idxbuf, 0, posbuf_a, lane, opad + r0)
        _pos_loop(idxbuf, HALF, posbuf_b, lane, cnt)

        pltpu.sync_copy(hid_ref.at[pl.ds(base, HALF)], rowbuf)
        pltpu.async_copy(rowbuf, xs_ref.at[posbuf_a], sem).wait()
        pltpu.sync_copy(hid_ref.at[pl.ds(base + HALF, HALF)], rowbuf)
        pltpu.async_copy(rowbuf, xs_ref.at[posbuf_b], sem).wait()
        pltpu.async_copy(gbuf_a, gs_ref.at[posbuf_a], sem).wait()
        pltpu.async_copy(gbuf_b, gs_ref.at[posbuf_b], sem).wait()
        pltpu.sync_copy(posbuf_a, pos_ref.at[pl.ds(base, HALF)])
        pltpu.sync_copy(posbuf_b, pos_ref.at[pl.ds(base + HALF, HALF)])

        @pl.when(wid == 0)
        def _():
            _tile_experts(opad, tebuf)
            pltpu.sync_copy(tebuf, te_ref)

    xs, gs, pos2, te, _ = k(hidden, idx2, gate2)
    return xs, gs, pos2, te


def _gmm_router_body(te_ref, xs_ref, gs_ref, We_ref, be_ref, wg_ref,
                     ys_ref, g3_ref, i3_ref):
    yraw = (
        jnp.dot(xs_ref[...].astype(jnp.bfloat16), We_ref[0],
                preferred_element_type=jnp.float32)
        + be_ref[0]
    )
    ys = gs_ref[...] * yraw
    ys_ref[...] = ys
    logits = jnp.dot(ys, wg_ref[...], preferred_element_type=jnp.float32)
    gate, idx = _router(logits)
    g3_ref[...] = gate
    i3_ref[...] = idx


def _gmm_router(te, xs, gs, We, be, wg):
    wgp = jnp.pad(wg, ((0, 0), (0, EPAD - E)))
    return pl.pallas_call(
        _gmm_router_body,
        grid_spec=pltpu.PrefetchScalarGridSpec(
            num_scalar_prefetch=1,
            grid=(XT,),
            in_specs=[
                pl.BlockSpec((TILE, D), lambda t, te: (t, 0)),
                pl.BlockSpec((TILE, 1), lambda t, te: (t, 0)),
                pl.BlockSpec((1, D, D), lambda t, te: (te[t], 0, 0)),
                pl.BlockSpec((1, 1, D), lambda t, te: (te[t], 0, 0)),
                pl.BlockSpec((D, EPAD), lambda t, te: (0, 0)),
            ],
            out_specs=[
                pl.BlockSpec((TILE, D), lambda t, te: (t, 0)),
                pl.BlockSpec((TILE, 1), lambda t, te: (t, 0)),
                pl.BlockSpec((TILE, 1), lambda t, te: (t, 0)),
            ],
        ),
        out_shape=[
            jax.ShapeDtypeStruct((TP, D), jnp.float32),
            jax.ShapeDtypeStruct((TP, 1), jnp.float32),
            jax.ShapeDtypeStruct((TP, 1), jnp.int32),
        ],
    )(te, xs, gs.reshape(TP, 1), We.astype(jnp.bfloat16),
      be.reshape(E, 1, D), wgp)


def _route3(ys, i3, g3, pos2):
    @functools.partial(
        pl.kernel,
        mesh=_sc_mesh(),
        out_type=[
            jax.ShapeDtypeStruct((TP, D), jnp.float32),
            jax.ShapeDtypeStruct((TP,), jnp.float32),
            jax.ShapeDtypeStruct((48,), jnp.int32),
            jax.ShapeDtypeStruct((48,), jnp.int32),
            jax.ShapeDtypeStruct((NW, 16), jnp.int32),
        ],
        scratch_types=[
            pltpu.VMEM((CH,), jnp.int32),
            pltpu.VMEM((HALF,), jnp.int32),
            pltpu.VMEM((HALF,), jnp.int32),
            pltpu.VMEM((HALF,), jnp.int32),
            pltpu.VMEM((HALF,), jnp.int32),
            pltpu.VMEM((HALF,), jnp.float32),
            pltpu.VMEM((HALF,), jnp.float32),
            pltpu.VMEM((HALF, D), jnp.float32),
            pltpu.VMEM((HALF,), jnp.int32),
            pltpu.VMEM((HALF,), jnp.int32),
            pltpu.VMEM((16,), jnp.int32),
            pltpu.VMEM((NW, 16), jnp.int32),
            pltpu.VMEM((48,), jnp.int32),
            pltpu.VMEM((48,), jnp.int32),
            pltpu.SemaphoreType.DMA,
        ],
    )
    def k(ys_ref, i3_ref, g3_ref, pos2_ref, xs3_ref, gs3_ref, te_ref,
          bnd_ref, hx_ref, pbuf, pbuf_a, pbuf_b, ivals_a, ivals_b,
          gvals_a, gvals_b, rowbuf, posbuf_a, posbuf_b, histv, histall,
          tebuf, bndbuf, sem):
        wid = lax.axis_index("s")
        base = wid * CH
        lane = lax.iota(jnp.int32, 16)
        zero = jnp.zeros((16,), jnp.int32)
        pltpu.sync_copy(pos2_ref.at[pl.ds(base, CH)], pbuf)
        for kk in range(HALF // 16):
            pbuf_a[pl.ds(kk * 16, 16)] = pbuf[pl.ds(kk * 16, 16)]
            pbuf_b[pl.ds(kk * 16, 16)] = pbuf[pl.ds(HALF + kk * 16, 16)]
        pltpu.async_copy(i3_ref.at[pbuf_a], ivals_a, sem).wait()
        pltpu.async_copy(i3_ref.at[pbuf_b], ivals_b, sem).wait()
        pltpu.async_copy(g3_ref.at[pbuf_a], gvals_a, sem).wait()
        pltpu.async_copy(g3_ref.at[pbuf_b], gvals_b, sem).wait()

        hist = _hist_loop(ivals_a, 0, lane, zero)
        histv[...] = _hist_loop(ivals_b, 0, lane, hist)
        pltpu.sync_copy(histv, hx_ref.at[wid])
        plsc.subcore_barrier()
        pltpu.sync_copy(hx_ref, histall)

        tot, c0, opad, r0 = _offsets(histall, wid, lane,
                                     batch_split=NW // 2)
        cnt = _pos_loop(ivals_a, 0, posbuf_a, lane, opad + r0)
        _pos_loop(ivals_b, 0, posbuf_b, lane, cnt)

        pltpu.async_copy(ys_ref.at[pbuf_a], rowbuf, sem).wait()
        pltpu.async_copy(rowbuf, xs3_ref.at[posbuf_a], sem).wait()
        pltpu.async_copy(ys_ref.at[pbuf_b], rowbuf, sem).wait()
        pltpu.async_copy(rowbuf, xs3_ref.at[posbuf_b], sem).wait()
        pltpu.async_copy(gvals_a, gs3_ref.at[posbuf_a], sem).wait()
        pltpu.async_copy(gvals_b, gs3_ref.at[posbuf_b], sem).wait()

        @pl.when(wid == 0)
        def _():
            _tile_experts(opad, tebuf)
            pltpu.sync_copy(tebuf, te_ref)
            bndbuf[pl.ds(0, 16)] = opad
            bndbuf[pl.ds(16, 16)] = c0
            bndbuf[pl.ds(32, 16)] = tot
            pltpu.sync_copy(bndbuf, bnd_ref)

    xs3, gs3, te3, bnd, _ = k(ys, i3, g3, pos2)
    return xs3, gs3, te3, bnd


def _gmm_sum_body(teb_ref, xs_ref, gs_ref, We_ref, be_ref, es_ref):
    t = pl.program_id(0)
    yraw = (
        jnp.dot(xs_ref[...].astype(jnp.bfloat16), We_ref[0],
                preferred_element_type=jnp.float32)
        + be_ref[0]
    )
    ys = gs_ref[...] * yraw
    r = jax.lax.broadcasted_iota(jnp.int32, (TILE, 1), 0) + t * TILE
    m0 = jnp.zeros((TILE, 1), jnp.bool_)
    m1 = jnp.zeros((TILE, 1), jnp.bool_)
    for e in range(E):
        off = teb_ref[XTP + e]
        c0 = teb_ref[XTP + 16 + e]
        c = teb_ref[XTP + 32 + e]
        m0 = m0 | ((r >= off) & (r < off + c0))
        m1 = m1 | ((r >= off + c0) & (r < off + c))
    zero = jnp.zeros_like(ys)
    s0 = jnp.sum(jnp.where(m0, ys, zero), axis=0, keepdims=True)[None]
    s1 = jnp.sum(jnp.where(m1, ys, zero), axis=0, keepdims=True)[None]

    @pl.when(t == 0)
    def _():
        es_ref[...] = jnp.zeros_like(es_ref)

    es_ref[...] += jnp.concatenate([s0, s1], axis=0)


def _gmm_sum(te3, bnd, xs, gs, We, be):
    teb = jnp.concatenate([te3, bnd])
    return pl.pallas_call(
        _gmm_sum_body,
        grid_spec=pltpu.PrefetchScalarGridSpec(
            num_scalar_prefetch=1,
            grid=(XT,),
            in_specs=[
                pl.BlockSpec((TILE, D), lambda t, teb: (t, 0)),
                pl.BlockSpec((TILE, 1), lambda t, teb: (t, 0)),
                pl.BlockSpec((1, D, D), lambda t, teb: (teb[t], 0, 0)),
                pl.BlockSpec((1, 1, D), lambda t, teb: (teb[t], 0, 0)),
            ],
            out_specs=pl.BlockSpec((B, 1, D), lambda t, teb: (0, 0, 0)),
        ),
        out_shape=jax.ShapeDtypeStruct((B, 1, D), jnp.float32),
    )(teb, xs, gs.reshape(TP, 1), We.astype(jnp.bfloat16),
      be.reshape(E, 1, D))


def _final_body(hs_ref, es_ref, y_ref, loss_ref):
    emb = (hs_ref[:, 0, :] + es_ref[:, 0, :]) / S
    m = jnp.max(emb, axis=1, keepdims=True)
    lse = m + jnp.log(jnp.sum(jnp.exp(emb - m), axis=1, keepdims=True))
    logp = emb - lse
    col = jax.lax.broadcasted_iota(jnp.int32, logp.shape, 1)
    v0 = jnp.sum(jnp.where(col[0:1] == y_ref[0], logp[0:1], 0.0))
    v1 = jnp.sum(jnp.where(col[1:2] == y_ref[1], logp[1:2], 0.0))
    loss_ref[0, 0] = -(v0 + v1) / B


def _final(hsums, esums, y):
    out = pl.pallas_call(
        _final_body,
        in_specs=[
            pl.BlockSpec((B, 1, D), lambda: (0, 0, 0)),
            pl.BlockSpec((B, 1, D), lambda: (0, 0, 0)),
            pl.BlockSpec(memory_space=pltpu.SMEM),
        ],
        out_specs=pl.BlockSpec(memory_space=pltpu.SMEM),
        out_shape=jax.ShapeDtypeStruct((1, 1), jnp.float32),
    )(hsums, esums, y)
    return out.reshape(())


def kernel(x, y, W1, b1, wg2, We2, be2, wg3, We3, be3):
    xt = x.reshape(T, D)
    hidden, g2, i2, hsums = _dense_router(xt.astype(jnp.bfloat16),
                                          W1.T.astype(jnp.bfloat16), b1, wg2)
    xs, gs, pos2, te = _route2(hidden, i2.reshape(T), g2.reshape(T))
    ys, g3, i3 = _gmm_router(te, xs, gs, We2, be2, wg3)
    xs3, gs3, te3, bnd = _route3(ys, i3.reshape(TP), g3.reshape(TP), pos2)
    esums = _gmm_sum(te3, bnd, xs3, gs3, We3, be3)
    return _final(hsums, esums, y.astype(jnp.int32))
